# Initial kernel scaffold; baseline (speedup 1.0000x reference)
import jax
import jax.numpy as jnp
from jax.experimental import pallas as pl


def kernel(inputs, lookup_table):
    raise NotImplementedError("write your pallas kernel here")


# speedup vs baseline: 2.5250x; 2.5250x over previous
import functools

import jax
import jax.numpy as jnp
from jax import lax
from jax.experimental import pallas as pl
from jax.experimental.pallas import tpu as pltpu
from jax.experimental.pallas import tpu_sc as plsc

_D = 128
_SCALE = float(_D) ** 0.5
_NC = 2
_NS = 16
_NW = _NC * _NS
_CHUNK = 128
_NBUF = 5


def _scale_body(t_ref, o_ref):
    o_ref[...] = t_ref[...] * _SCALE


def _scale_table(table):
    rows, d = table.shape
    blk = 2000
    assert rows % blk == 0
    return pl.pallas_call(
        _scale_body,
        grid=(rows // blk,),
        in_specs=[pl.BlockSpec((blk, d), lambda i: (i, 0))],
        out_specs=pl.BlockSpec((blk, d), lambda i: (i, 0)),
        out_shape=jax.ShapeDtypeStruct((rows, d), table.dtype),
    )(table)


@functools.partial(jax.jit, static_argnums=(2,))
def _sc_gather(table, idx3, nchunk):
    b_total = _NW * nchunk * _CHUNK
    mesh = plsc.VectorSubcoreMesh(
        core_axis_name="c", subcore_axis_name="s",
        num_cores=_NC, num_subcores=_NS)
    per_w = nchunk * _CHUNK

    @functools.partial(
        pl.kernel,
        out_type=jax.ShapeDtypeStruct((b_total, _D), jnp.float32),
        mesh=mesh,
        scratch_types=(
            [pltpu.VMEM((nchunk, _CHUNK), jnp.int32)]
            + [pltpu.VMEM((_CHUNK, _D), jnp.float32) for _ in range(_NBUF)]
            + [pltpu.SemaphoreType.DMA for _ in range(2 * _NBUF)]
        ),
    )
    def gather_kernel(table_hbm, idx_hbm, out_hbm, idx_v, *rest):
        bufs = rest[:_NBUF]
        gsems = rest[_NBUF:2 * _NBUF]
        ssems = rest[2 * _NBUF:]
        wid = lax.axis_index("s") * _NC + lax.axis_index("c")
        base = wid * per_w
        pltpu.sync_copy(idx_hbm.at[wid], idx_v)

        def out_slice(g):
            return out_hbm.at[pl.ds(base + g * _CHUNK, _CHUNK)]

        for j in range(_NBUF - 2):
            pltpu.async_copy(table_hbm.at[idx_v.at[j]], bufs[j], gsems[j])

        def slot(g, j):
            k = (j - 2) % _NBUF

            @pl.when(g >= 2)
            def _():
                pltpu.make_async_copy(bufs[k], out_slice(g - 2),
                                      ssems[k]).wait()

            @pl.when(g + _NBUF - 2 < nchunk)
            def _():
                pltpu.async_copy(table_hbm.at[idx_v.at[g + _NBUF - 2]],
                                 bufs[k], gsems[k])

            pltpu.make_async_copy(table_hbm.at[idx_v.at[g]], bufs[j],
                                  gsems[j]).wait()
            pltpu.async_copy(bufs[j], out_slice(g), ssems[j])

        def round_body(r, carry):
            g0 = r * _NBUF
            for j in range(_NBUF):
                slot(g0 + j, j)
            return carry

        lax.fori_loop(0, nchunk // _NBUF, round_body, 0)

        for g in (nchunk - 2, nchunk - 1):
            pltpu.make_async_copy(bufs[g % _NBUF], out_slice(g),
                                  ssems[g % _NBUF]).wait()

    return gather_kernel(table, idx3)


def kernel(inputs, lookup_table):
    orig_shape = inputs.shape
    idx = inputs.reshape(-1).astype(jnp.int32)
    n = idx.shape[0]
    assert n % (_NW * _CHUNK) == 0
    nchunk = n // (_NW * _CHUNK)
    assert nchunk % _NBUF == 0 and nchunk >= 2 * _NBUF
    table_scaled = _scale_table(lookup_table)
    idx3 = idx.reshape(_NW, nchunk, _CHUNK)
    out = _sc_gather(table_scaled, idx3, nchunk)
    return out.reshape(*orig_shape, _D)


# speedup vs baseline: 4.0106x; 1.5884x over previous
import functools

import jax
import jax.numpy as jnp
from jax import lax
from jax.experimental import pallas as pl
from jax.experimental.pallas import tpu as pltpu
from jax.experimental.pallas import tpu_sc as plsc

_D = 128
_SCALE = float(_D) ** 0.5
_NC = 2
_NS = 16
_NW = _NC * _NS
_NBUF = 8
_K = 3


def _scale_body(t_ref, o_ref):
    o_ref[...] = t_ref[...] * _SCALE


def _scale_table(table):
    rows, d = table.shape
    blk = 2000
    assert rows % blk == 0
    return pl.pallas_call(
        _scale_body,
        grid=(rows // blk,),
        in_specs=[pl.BlockSpec((blk, d), lambda i: (i, 0))],
        out_specs=pl.BlockSpec((blk, d), lambda i: (i, 0)),
        out_shape=jax.ShapeDtypeStruct((rows, d), table.dtype),
    )(table)


@functools.partial(jax.jit, static_argnums=(2, 3))
def _sc_gather(table, idx3, nchunk, seq):
    nbatch = _NW * nchunk
    mesh = plsc.VectorSubcoreMesh(
        core_axis_name="c", subcore_axis_name="s",
        num_cores=_NC, num_subcores=_NS)

    @functools.partial(
        pl.kernel,
        out_type=jax.ShapeDtypeStruct((nbatch, seq, _D), jnp.float32),
        mesh=mesh,
        compiler_params=pltpu.CompilerParams(use_tc_tiling_on_sc=True),
        scratch_types=(
            [pltpu.VMEM((nchunk, seq), jnp.int32)]
            + [pltpu.VMEM((seq, _D), jnp.float32) for _ in range(_NBUF)]
            + [pltpu.SemaphoreType.DMA for _ in range(2 * _NBUF)]
        ),
    )
    def gather_kernel(table_hbm, idx_hbm, out_hbm, idx_v, *rest):
        bufs = rest[:_NBUF]
        gsems = rest[_NBUF:2 * _NBUF]
        ssems = rest[2 * _NBUF:]
        wid = lax.axis_index("s") * _NC + lax.axis_index("c")
        wb0 = wid * nchunk
        pltpu.sync_copy(idx_hbm.at[wid], idx_v)

        for j in range(_NBUF - _K):
            pltpu.async_copy(table_hbm.at[idx_v.at[j]], bufs[j], gsems[j])

        def slot(g, j):
            k = (j - _K) % _NBUF

            @pl.when(g >= _K)
            def _():
                pltpu.make_async_copy(bufs[k], out_hbm.at[wb0 + g - _K],
                                      ssems[k]).wait()

            @pl.when(g + _NBUF - _K < nchunk)
            def _():
                pltpu.async_copy(table_hbm.at[idx_v.at[g + _NBUF - _K]],
                                 bufs[k], gsems[k])

            pltpu.make_async_copy(table_hbm.at[idx_v.at[g]], bufs[j],
                                  gsems[j]).wait()
            pltpu.async_copy(bufs[j], out_hbm.at[wb0 + g], ssems[j])

        def round_body(r, carry):
            g0 = r * _NBUF
            for j in range(_NBUF):
                slot(g0 + j, j)
            return carry

        lax.fori_loop(0, nchunk // _NBUF, round_body, 0)

        for g in range(nchunk - _K, nchunk):
            pltpu.make_async_copy(bufs[g % _NBUF], out_hbm.at[wb0 + g],
                                  ssems[g % _NBUF]).wait()

    return gather_kernel(table, idx3)


def kernel(inputs, lookup_table):
    nbatch, seq = inputs.shape
    idx = inputs.astype(jnp.int32)
    assert nbatch % _NW == 0
    nchunk = nbatch // _NW
    assert nchunk % _NBUF == 0 and nchunk >= 2 * _NBUF
    table_scaled = _scale_table(lookup_table)
    idx3 = idx.reshape(_NW, nchunk, seq)
    return _sc_gather(table_scaled, idx3, nchunk, seq)


# speedup vs baseline: 6.0157x; 1.4999x over previous
import functools

import jax
import jax.numpy as jnp
from jax import lax
from jax.experimental import pallas as pl
from jax.experimental.pallas import tpu as pltpu
from jax.experimental.pallas import tpu_sc as plsc

_D = 128
_SCALE = float(_D) ** 0.5
_NC = 2
_NS = 16
_NW = _NC * _NS
_CHUNK = 128
_NBUF = 5
_K = 2


def _scale_body(t_ref, o_ref):
    o_ref[...] = t_ref[...] * _SCALE


def _scale_table(table):
    rows, d = table.shape
    blk = 2000
    assert rows % blk == 0
    return pl.pallas_call(
        _scale_body,
        grid=(rows // blk,),
        in_specs=[pl.BlockSpec((blk, d), lambda i: (i, 0))],
        out_specs=pl.BlockSpec((blk, d), lambda i: (i, 0)),
        out_shape=jax.ShapeDtypeStruct((rows, d), table.dtype),
    )(table)


@functools.partial(jax.jit, static_argnums=(2,))
def _sc_gather(table, idx3, nchunk):
    b_total = _NW * nchunk * _CHUNK
    mesh = plsc.VectorSubcoreMesh(
        core_axis_name="c", subcore_axis_name="s",
        num_cores=_NC, num_subcores=_NS)
    per_w = nchunk * _CHUNK

    @functools.partial(
        pl.kernel,
        out_type=jax.ShapeDtypeStruct((b_total, _D), jnp.float32),
        mesh=mesh,
        compiler_params=pltpu.CompilerParams(use_tc_tiling_on_sc=True),
        scratch_types=(
            [pltpu.VMEM((nchunk, _CHUNK), jnp.int32)]
            + [pltpu.VMEM((_CHUNK, _D), jnp.float32) for _ in range(_NBUF)]
            + [pltpu.SemaphoreType.DMA for _ in range(2 * _NBUF)]
        ),
    )
    def gather_kernel(table_hbm, idx_hbm, out_hbm, idx_v, *rest):
        bufs = rest[:_NBUF]
        gsems = rest[_NBUF:2 * _NBUF]
        ssems = rest[2 * _NBUF:]
        wid = lax.axis_index("s") * _NC + lax.axis_index("c")
        base = wid * per_w
        pltpu.sync_copy(idx_hbm.at[wid], idx_v)

        def out_slice(g):
            return out_hbm.at[pl.ds(base + g * _CHUNK, _CHUNK)]

        for j in range(_NBUF - _K):
            pltpu.async_copy(table_hbm.at[idx_v.at[j]], bufs[j], gsems[j])

        def slot(g, j):
            k = (j - _K) % _NBUF

            @pl.when(g >= _K)
            def _():
                pltpu.make_async_copy(bufs[k], out_slice(g - _K),
                                      ssems[k]).wait()

            @pl.when(g + _NBUF - _K < nchunk)
            def _():
                pltpu.async_copy(table_hbm.at[idx_v.at[g + _NBUF - _K]],
                                 bufs[k], gsems[k])

            pltpu.make_async_copy(table_hbm.at[idx_v.at[g]], bufs[j],
                                  gsems[j]).wait()
            pltpu.async_copy(bufs[j], out_slice(g), ssems[j])

        def round_body(r, carry):
            g0 = r * _NBUF
            for j in range(_NBUF):
                slot(g0 + j, j)
            return carry

        lax.fori_loop(0, nchunk // _NBUF, round_body, 0)

        for g in range(nchunk - _K, nchunk):
            pltpu.make_async_copy(bufs[g % _NBUF], out_slice(g),
                                  ssems[g % _NBUF]).wait()

    return gather_kernel(table, idx3)


def kernel(inputs, lookup_table):
    nbatch, seq = inputs.shape
    idx = inputs.T.reshape(-1).astype(jnp.int32)
    n = idx.shape[0]
    assert n % (_NW * _CHUNK) == 0
    nchunk = n // (_NW * _CHUNK)
    assert nchunk % _NBUF == 0 and nchunk >= 2 * _NBUF
    table_scaled = _scale_table(lookup_table)
    idx3 = idx.reshape(_NW, nchunk, _CHUNK)
    out = _sc_gather(table_scaled, idx3, nchunk)
    return out.reshape(seq, nbatch, _D).transpose(1, 0, 2)


# speedup vs baseline: 9.1758x; 1.5253x over previous
import functools

import jax
import jax.numpy as jnp
from jax import lax
from jax.experimental import pallas as pl
from jax.experimental.pallas import tpu as pltpu
from jax.experimental.pallas import tpu_sc as plsc

_D = 128
_SCALE = float(_D) ** 0.5
_NC = 2
_NS = 16
_NW = _NC * _NS
_CHUNK = 128
_NBUF = 5
_K = 2


@functools.partial(jax.jit, static_argnums=(2,))
def _sc_gather(table, idx3, nchunk):
    b_total = _NW * nchunk * _CHUNK
    mesh = plsc.VectorSubcoreMesh(
        core_axis_name="c", subcore_axis_name="s",
        num_cores=_NC, num_subcores=_NS)
    per_w = nchunk * _CHUNK

    @functools.partial(
        pl.kernel,
        out_type=jax.ShapeDtypeStruct((b_total, _D), jnp.float32),
        mesh=mesh,
        compiler_params=pltpu.CompilerParams(use_tc_tiling_on_sc=True),
        scratch_types=(
            [pltpu.VMEM((nchunk, _CHUNK), jnp.int32)]
            + [pltpu.VMEM((_CHUNK, _D), jnp.float32) for _ in range(_NBUF)]
            + [pltpu.SemaphoreType.DMA for _ in range(2 * _NBUF)]
        ),
    )
    def gather_kernel(table_hbm, idx_hbm, out_hbm, idx_v, *rest):
        bufs = rest[:_NBUF]
        gsems = rest[_NBUF:2 * _NBUF]
        ssems = rest[2 * _NBUF:]
        wid = lax.axis_index("s") * _NC + lax.axis_index("c")
        base = wid * per_w
        pltpu.sync_copy(idx_hbm.at[wid], idx_v)

        def out_slice(g):
            return out_hbm.at[pl.ds(base + g * _CHUNK, _CHUNK)]

        for j in range(_NBUF - _K):
            pltpu.async_copy(table_hbm.at[idx_v.at[j]], bufs[j], gsems[j])

        def slot(g, j):
            k = (j - _K) % _NBUF

            @pl.when(g >= _K)
            def _():
                pltpu.make_async_copy(bufs[k], out_slice(g - _K),
                                      ssems[k]).wait()

            @pl.when(g + _NBUF - _K < nchunk)
            def _():
                pltpu.async_copy(table_hbm.at[idx_v.at[g + _NBUF - _K]],
                                 bufs[k], gsems[k])

            pltpu.make_async_copy(table_hbm.at[idx_v.at[g]], bufs[j],
                                  gsems[j]).wait()
            buf = bufs[j]

            @plsc.parallel_loop(0, _CHUNK, unroll=4)
            def _(r):
                for v in range(_D // 16):
                    sl = (r, pl.ds(v * 16, 16))
                    buf[sl] = buf[sl] * _SCALE

            pltpu.async_copy(bufs[j], out_slice(g), ssems[j])

        def round_body(r, carry):
            g0 = r * _NBUF
            for j in range(_NBUF):
                slot(g0 + j, j)
            return carry

        lax.fori_loop(0, nchunk // _NBUF, round_body, 0)

        for g in range(nchunk - _K, nchunk):
            pltpu.make_async_copy(bufs[g % _NBUF], out_slice(g),
                                  ssems[g % _NBUF]).wait()

    return gather_kernel(table, idx3)


def kernel(inputs, lookup_table):
    nbatch, seq = inputs.shape
    idx = inputs.T.reshape(-1).astype(jnp.int32)
    n = idx.shape[0]
    assert n % (_NW * _CHUNK) == 0
    nchunk = n // (_NW * _CHUNK)
    assert nchunk % _NBUF == 0 and nchunk >= 2 * _NBUF
    idx3 = idx.reshape(_NW, nchunk, _CHUNK)
    out = _sc_gather(lookup_table, idx3, nchunk)
    return out.reshape(seq, nbatch, _D).transpose(1, 0, 2)
